# baseline (speedup 1.0000x reference)
import functools
import math

import jax
import jax.numpy as jnp
from jax import lax
from jax.experimental import pallas as pl
from jax.experimental.pallas import tpu as pltpu
from jax.experimental.pallas import tpu_sc as plsc

N_NODES = 10000
N_EDGES = 160000
BATCH = 256
NODE_DIM = 128
EDGE_DIM = 16
HID = 256
N_LAYERS = 4
HEADS = 8
OUTC = HID // HEADS
MAX_T = 1000

NCORES = 2
NSUB = 16
CH = 128
GROUPS = CH // 16
EPT = 5120
NCHUNK = EPT // CH
EPAD = NCORES * NSUB * EPT
QW = 64
NPAD = 10240
AROWS = NPAD // 2
ROWS_PT = AROWS // NSUB
DROWS = NPAD * 2 // 128

_mesh = plsc.VectorSubcoreMesh(core_axis_name="c", subcore_axis_name="s")


def _make_edge_quarter(qoff):
  @functools.partial(
      pl.kernel,
      out_type=(jax.ShapeDtypeStruct((NCORES, AROWS, 128), jnp.float32),
                jax.ShapeDtypeStruct((NCORES, DROWS, 128), jnp.float32)),
      mesh=_mesh,
      compiler_params=pltpu.CompilerParams(needs_layout_passes=False),
      scratch_types=[
          pltpu.VMEM((CH,), jnp.int32),
          pltpu.VMEM((CH,), jnp.int32),
          pltpu.VMEM((CH,), jnp.int32),
          pltpu.VMEM((CH,), jnp.int32),
          pltpu.VMEM((CH,), jnp.int32),
          pltpu.VMEM((CH, 128), jnp.float32),
          pltpu.VMEM((CH, 128), jnp.float32),
          pltpu.VMEM((EDGE_DIM, 128), jnp.float32),
          pltpu.VMEM((128,), jnp.float32),
          pltpu.VMEM((CH, 128), jnp.float32),
          pltpu.VMEM((CH, 128), jnp.float32),
          pltpu.VMEM_SHARED((AROWS, 128), jnp.float32),
          pltpu.VMEM_SHARED((DROWS, 128), jnp.float32),
          pltpu.SemaphoreType.DMA,
          pltpu.SemaphoreType.DMA,
      ],
  )
  def _edge_quarter(xlh, xrh, tab, att, srcp, dstp, eip, out_hbm, den_hbm,
                    sidx, didx, didxa, didxd, eidx, xlr, xrr, tabv, attv,
                    orows, drows, acc, dacc, sem1, sem2):
    core = lax.axis_index("c")
    sub = lax.axis_index("s")
    lanes = jnp.arange(16, dtype=jnp.int32)

    pltpu.sync_copy(tab, tabv)
    pltpu.sync_copy(att, attv)

    def zrow(i, _):
        def zcol(j, _):
            orows[i, pl.ds(j * 16, 16)] = jnp.zeros((16,), jnp.float32)
            drows[i, pl.ds(j * 16, 16)] = jnp.zeros((16,), jnp.float32)
            return 0
        return lax.fori_loop(0, 128 // 16, zcol, 0)
    lax.fori_loop(0, CH, zrow, 0)

    row0 = sub * ROWS_PT
    pltpu.sync_copy(orows, acc.at[pl.ds(row0, CH)])
    pltpu.sync_copy(orows, acc.at[pl.ds(row0 + CH, CH)])
    pltpu.sync_copy(orows.at[pl.ds(0, 64)], acc.at[pl.ds(row0 + 2 * CH, 64)])
    @pl.when(sub < 5)
    def _():
        pltpu.sync_copy(orows.at[pl.ds(0, 32)], dacc.at[pl.ds(sub * 32, 32)])
    plsc.subcore_barrier()

    ebase = (core * NSUB + sub) * EPT

    def chunk_body(ci, _):
        base = ebase + ci * CH
        pltpu.sync_copy(srcp.at[pl.ds(base, CH)], sidx)
        pltpu.sync_copy(dstp.at[pl.ds(base, CH)], didx)
        pltpu.sync_copy(eip.at[pl.ds(base, CH)], eidx)
        cp1 = pltpu.async_copy(xlh.at[sidx], xlr, sem1)
        cp2 = pltpu.async_copy(xrh.at[didx], xrr, sem2)
        def dbody(g, _):
            gl = lanes + g * 16
            dv = plsc.load_gather(didx, [gl])
            plsc.store_scatter(didxa, [gl], lax.shift_right_logical(dv, 1))
            plsc.store_scatter(didxd, [gl], lax.shift_right_logical(dv, 6))
            return 0
        lax.fori_loop(0, GROUPS, dbody, 0)
        cp1.wait()
        cp2.wait()

        zv = jnp.zeros((16,), jnp.float32)
        gls = [lanes + g * 16 for g in range(GROUPS)]
        eivs = [plsc.load_gather(eidx, [gls[g]]) for g in range(GROUPS)]
        dstvs = [plsc.load_gather(didx, [gls[g]]) for g in range(GROUPS)]
        mskfs = [jnp.where(plsc.load_gather(sidx, [gls[g]]) != dstvs[g],
                           1.0, 0.0).astype(jnp.float32) for g in range(GROUPS)]
        obases = [jnp.bitwise_and(dstvs[g], 1) * QW for g in range(GROUPS)]
        dcol0s = [jnp.bitwise_and(dstvs[g], 63) * 2 for g in range(GROUPS)]

        for h in range(2):
            col0 = jnp.full((16,), qoff + h * 32, jnp.int32)

            def abody(c, accs):
                colv = col0 + c
                av = plsc.load_gather(attv, [colv])
                out = []
                for g in range(GROUPS):
                    xlv = plsc.load_gather(xlr, [gls[g], colv])
                    xrv = plsc.load_gather(xrr, [gls[g], colv])
                    tbv = plsc.load_gather(tabv, [eivs[g], colv])
                    s = xlv + xrv + tbv
                    out.append(accs[g] + jnp.maximum(s, 0.2 * s) * av)
                return tuple(out)
            accs = plsc.parallel_loop(
                0, 32, unroll=2,
                carry=tuple(jnp.zeros((16,), jnp.float32)
                            for _ in range(GROUPS)))(abody)
            exs = [jnp.exp(accs[g]) * mskfs[g] for g in range(GROUPS)]
            rel0 = jnp.full((16,), h * 32, jnp.int32)

            def obody(c):
                colv = col0 + c
                relv = rel0 + c
                for g in range(GROUPS):
                    xlv = plsc.load_gather(xlr, [gls[g], colv])
                    oc = obases[g] + relv
                    plsc.store_scatter(orows, [gls[g], oc], xlv * exs[g])
                    plsc.store_scatter(orows, [gls[g], jnp.bitwise_xor(oc, QW)], zv)
            plsc.parallel_loop(0, 32, unroll=2)(obody)
            for g in range(GROUPS):
                plsc.store_scatter(drows, [gls[g], dcol0s[g] + h], exs[g])

        pltpu.sync_copy(orows, acc.at[didxa], add=True)
        pltpu.sync_copy(drows, dacc.at[didxd], add=True)

        def zbody(g, _):
            gl = lanes + g * 16
            dv = plsc.load_gather(didx, [gl])
            z = jnp.zeros((16,), jnp.float32)
            c0 = jnp.bitwise_and(dv, 63) * 2
            for h in range(2):
                plsc.store_scatter(drows, [gl, c0 + h], z)
            return 0
        lax.fori_loop(0, GROUPS, zbody, 0)
        return 0

    lax.fori_loop(0, NCHUNK, chunk_body, 0)
    plsc.subcore_barrier()

    pltpu.sync_copy(acc.at[pl.ds(row0, CH)], out_hbm.at[core, pl.ds(row0, CH)])
    pltpu.sync_copy(acc.at[pl.ds(row0 + CH, CH)],
                    out_hbm.at[core, pl.ds(row0 + CH, CH)])
    pltpu.sync_copy(acc.at[pl.ds(row0 + 2 * CH, 64)],
                    out_hbm.at[core, pl.ds(row0 + 2 * CH, 64)])
    @pl.when(sub < 5)
    def _():
        pltpu.sync_copy(dacc.at[pl.ds(sub * 32, 32)],
                        den_hbm.at[core, pl.ds(sub * 32, 32)])

  return _edge_quarter


_edge_quarter_lo = _make_edge_quarter(0)
_edge_quarter_hi = _make_edge_quarter(64)


def _layernorm(h, g, b):
    mu = jnp.mean(h, axis=-1, keepdims=True)
    var = jnp.var(h, axis=-1, keepdims=True)
    return (h - mu) / jnp.sqrt(var + 1e-5) * g + b


def _time_embed(t):
    half = HID // 2
    tf = jnp.clip(t, 0, 999).astype(jnp.float32)
    freqs = jnp.exp(jnp.arange(half, dtype=jnp.float32) * -(math.log(10000.0) / (half - 1)))
    ang = tf[:, None] * freqs[None, :]
    return jnp.concatenate([jnp.sin(ang), jnp.cos(ang)], axis=-1)


def _gatv2(xn, srcp, dstp, eip, e_tab, loop_e, p):
    xl = xn @ p["Wl"] + p["bl"]
    xr = xn @ p["Wr"] + p["br"]
    att = p["att"].reshape(-1)

    num_q = []
    ex_q = []
    for q in range(4):
        hh = q // 2
        sl = slice(hh * 128, (hh + 1) * 128)
        fn = _edge_quarter_lo if q % 2 == 0 else _edge_quarter_hi
        out, den = fn(xl[:, sl], xr[:, sl], e_tab[:, sl], att[sl],
                      srcp, dstp, eip)
        num_q.append((out[0] + out[1]).reshape(NPAD, QW)[:N_NODES])
        ex_q.append((den[0] + den[1]).reshape(-1)[:N_NODES * 2].reshape(N_NODES, 2))
    num = jnp.concatenate(num_q, axis=-1).reshape(-1, HEADS, OUTC)
    ex_sum = jnp.concatenate(ex_q, axis=-1)

    xl3 = xl.reshape(-1, HEADS, OUTC)
    xr3 = xr.reshape(-1, HEADS, OUTC)
    m_loop = jax.nn.leaky_relu(xl3 + xr3 + loop_e, 0.2)
    a_loop = jnp.sum(m_loop * p["att"][None], axis=-1)
    ex_loop = jnp.exp(a_loop)
    den = ex_sum + ex_loop
    out = (num + xl3 * ex_loop[..., None]) / den[..., None]
    return out.reshape(-1, HID) + p["bias"]


def kernel(x, edge_index, edge_attr, batch, timesteps, params):
    xi = jnp.clip(x, 0, NODE_DIM - 1)
    ei = jnp.clip(edge_attr, 0, EDGE_DIM - 1).astype(jnp.int32)
    node = params["node_embed"][xi]
    te = _time_embed(timesteps)
    te = jax.nn.gelu(te @ params["tm_W1"] + params["tm_b1"], approximate=False) @ params["tm_W2"] + params["tm_b2"]
    node = node + te[batch]
    src = edge_index[0].astype(jnp.int32)
    dst = edge_index[1].astype(jnp.int32)

    pad = EPAD - N_EDGES
    srcp = jnp.concatenate([src, jnp.zeros((pad,), jnp.int32)])
    dstp = jnp.concatenate([dst, jnp.zeros((pad,), jnp.int32)])
    eip = jnp.concatenate([ei, jnp.zeros((pad,), jnp.int32)])

    mask = (src != dst)
    mf = mask.astype(jnp.float32)
    onehot = jax.nn.one_hot(ei, EDGE_DIM, dtype=jnp.float32) * mf[:, None]
    cnt16 = jax.ops.segment_sum(onehot, dst, num_segments=N_NODES)
    cnt = jnp.sum(cnt16, axis=-1)
    emean = (cnt16 @ params["edge_embed"]) / jnp.maximum(cnt, 1.0)[:, None]

    for lp in params["layers"]:
        e_tab = params["edge_embed"] @ lp["We"]
        loop_e = (emean @ lp["We"]).reshape(-1, HEADS, OUTC)
        attn = _gatv2(node, srcp, dstp, eip, e_tab, loop_e, lp)
        node = _layernorm(node + attn, lp["n1g"], lp["n1b"])
        h = jax.nn.gelu(node @ lp["fW1"] + lp["fb1"], approximate=False) @ lp["fW2"] + lp["fb2"]
        node = _layernorm(node + h, lp["n2g"], lp["n2b"])

    pred_node = _layernorm(node, params["hn_g"], params["hn_b"]) @ params["hn_W"] + params["hn_bW"]
    ef = (node[src] + node[dst]) * 0.5
    pred_edge = _layernorm(ef, params["he_g"], params["he_b"]) @ params["he_W"] + params["he_bW"]
    return (pred_node, pred_edge)


# speedup vs baseline: 1.3721x; 1.3721x over previous
import functools
import math

import jax
import jax.numpy as jnp
from jax import lax
from jax.experimental import pallas as pl
from jax.experimental.pallas import tpu as pltpu
from jax.experimental.pallas import tpu_sc as plsc

N_NODES = 10000
N_EDGES = 160000
BATCH = 256
NODE_DIM = 128
EDGE_DIM = 16
HID = 256
N_LAYERS = 4
HEADS = 8
OUTC = HID // HEADS
MAX_T = 1000

NCORES = 2
NSUB = 16
CH = 64
GROUPS = CH // 16
BLK = 3 * CH
EPT = 5120
NCHUNK = EPT // CH
EPAD = NCORES * NSUB * EPT
NPAD = 10240
ROWS_PT = NPAD // NSUB
DROWS = NPAD * 4 // 128

_mesh = plsc.VectorSubcoreMesh(core_axis_name="c", subcore_axis_name="s")


@functools.partial(
    pl.kernel,
    out_type=(jax.ShapeDtypeStruct((NCORES, NPAD, 128), jnp.float32),
              jax.ShapeDtypeStruct((NCORES, DROWS, 128), jnp.float32)),
    mesh=_mesh,
    compiler_params=pltpu.CompilerParams(needs_layout_passes=False),
    scratch_types=[
        pltpu.VMEM((2 * BLK,), jnp.int32),
        pltpu.VMEM((CH,), jnp.int32),
        pltpu.VMEM((CH,), jnp.int32),
        pltpu.VMEM((2 * CH, 128), jnp.float32),
        pltpu.VMEM((CH, 128), jnp.float32),
        pltpu.VMEM((EDGE_DIM, 128), jnp.float32),
        pltpu.VMEM((128,), jnp.float32),
        pltpu.VMEM((CH, 128), jnp.float32),
        pltpu.VMEM((CH, 128), jnp.float32),
        pltpu.VMEM_SHARED((NPAD, 128), jnp.float32),
        pltpu.VMEM_SHARED((DROWS, 128), jnp.float32),
        pltpu.SemaphoreType.DMA,
        pltpu.SemaphoreType.DMA,
        pltpu.SemaphoreType.DMA,
    ],
)
def _edge_half(xlh, xrh, tab, att, edata, out_hbm, den_hbm,
               eidx3, didxa, didxd, xlr, xrr, tabv, attv,
               orows, drows, acc, dacc, semxl0, semxl1, semxr):
    core = lax.axis_index("c")
    sub = lax.axis_index("s")
    lanes = jnp.arange(16, dtype=jnp.int32)

    pltpu.sync_copy(tab, tabv)
    pltpu.sync_copy(att, attv)

    def zrow(i, _):
        def zcol(j, _):
            orows[i, pl.ds(j * 16, 16)] = jnp.zeros((16,), jnp.float32)
            drows[i, pl.ds(j * 16, 16)] = jnp.zeros((16,), jnp.float32)
            return 0
        return lax.fori_loop(0, 128 // 16, zcol, 0)
    lax.fori_loop(0, CH, zrow, 0)

    row0 = sub * ROWS_PT
    for k in range(ROWS_PT // CH):
        pltpu.sync_copy(orows, acc.at[pl.ds(row0 + k * CH, CH)])
    @pl.when(sub < 10)
    def _():
        pltpu.sync_copy(orows.at[pl.ds(0, 32)], dacc.at[pl.ds(sub * 32, 32)])
    plsc.subcore_barrier()

    tilebase = (core * NSUB + sub) * NCHUNK

    def _fetch_idx(ci, buf):
        g = (tilebase + ci) * BLK
        pltpu.sync_copy(edata.at[pl.ds(g, BLK)],
                        eidx3.at[pl.ds(buf * BLK, BLK)])

    def _issue_xl(buf, sem):
        return pltpu.async_copy(xlh.at[eidx3.at[pl.ds(buf * BLK, CH)]],
                                xlr.at[pl.ds(buf * CH, CH)], sem)

    def _desc_xl(buf, sem):
        return pltpu.make_async_copy(xlh.at[eidx3.at[pl.ds(buf * BLK, CH)]],
                                     xlr.at[pl.ds(buf * CH, CH)], sem)

    _fetch_idx(0, 0)
    _issue_xl(0, semxl0)

    def chunk_body(ci, _):
        b = jnp.bitwise_and(ci, 1)
        boff = b * BLK
        nxt = jnp.minimum(ci + 1, NCHUNK - 1)
        nb = jnp.bitwise_and(ci + 1, 1)
        _fetch_idx(nxt, nb)
        @pl.when(nb == 0)
        def _():
            _issue_xl(0, semxl0)
        @pl.when(nb == 1)
        def _():
            _issue_xl(1, semxl1)
        cxr = pltpu.async_copy(xrh.at[eidx3.at[pl.ds(boff + CH, CH)]],
                               xrr, semxr)

        def dbody(g, _):
            gl = lanes + g * 16
            dv = plsc.load_gather(eidx3, [boff + CH + gl])
            plsc.store_scatter(didxa, [gl], dv)
            plsc.store_scatter(didxd, [gl], lax.shift_right_logical(dv, 5))
            return 0
        lax.fori_loop(0, GROUPS, dbody, 0)

        @pl.when(b == 0)
        def _():
            _desc_xl(0, semxl0).wait()
        @pl.when(b == 1)
        def _():
            _desc_xl(1, semxl1).wait()
        cxr.wait()

        gls = [lanes + g * 16 for g in range(GROUPS)]
        bgls = [gls[g] + b * CH for g in range(GROUPS)]
        eivs = [plsc.load_gather(eidx3, [boff + 2 * CH + gls[g]])
                for g in range(GROUPS)]
        dstvs = [plsc.load_gather(eidx3, [boff + CH + gls[g]])
                 for g in range(GROUPS)]
        mskfs = [jnp.where(plsc.load_gather(eidx3, [boff + gls[g]]) != dstvs[g],
                           1.0, 0.0).astype(jnp.float32) for g in range(GROUPS)]
        dcol0s = [jnp.bitwise_and(dstvs[g], 31) * 4 for g in range(GROUPS)]

        for g in range(GROUPS):
            for h in range(4):
                col0 = jnp.full((16,), h * 32, jnp.int32)

                def abody(c, accv):
                    colv = col0 + c
                    xlv = plsc.load_gather(xlr, [bgls[g], colv])
                    xrv = plsc.load_gather(xrr, [gls[g], colv])
                    tbv = plsc.load_gather(tabv, [eivs[g], colv])
                    av = plsc.load_gather(attv, [colv])
                    s = xlv + xrv + tbv
                    m = jnp.maximum(s, 0.2 * s)
                    return accv + m * av
                a_h = plsc.parallel_loop(
                    0, 32, unroll=8,
                    carry=jnp.zeros((16,), jnp.float32))(abody)
                ex_h = jnp.exp(a_h) * mskfs[g]

                def obody(c):
                    colv = col0 + c
                    xlv = plsc.load_gather(xlr, [bgls[g], colv])
                    plsc.store_scatter(orows, [gls[g], colv], xlv * ex_h)
                plsc.parallel_loop(0, 32, unroll=8)(obody)
                plsc.store_scatter(drows, [gls[g], dcol0s[g] + h], ex_h)

        pltpu.sync_copy(orows, acc.at[didxa], add=True)
        pltpu.sync_copy(drows, dacc.at[didxd], add=True)

        def zbody(g, _):
            gl = lanes + g * 16
            dv = plsc.load_gather(eidx3, [boff + CH + gl])
            z = jnp.zeros((16,), jnp.float32)
            c0 = jnp.bitwise_and(dv, 31) * 4
            for h in range(4):
                plsc.store_scatter(drows, [gl, c0 + h], z)
            return 0
        lax.fori_loop(0, GROUPS, zbody, 0)
        return 0

    lax.fori_loop(0, NCHUNK, chunk_body, 0)
    _desc_xl(0, semxl0).wait()
    plsc.subcore_barrier()

    for k in range(ROWS_PT // 128):
        pltpu.sync_copy(acc.at[pl.ds(row0 + k * 128, 128)],
                        out_hbm.at[core, pl.ds(row0 + k * 128, 128)])
    @pl.when(sub < 10)
    def _():
        pltpu.sync_copy(dacc.at[pl.ds(sub * 32, 32)],
                        den_hbm.at[core, pl.ds(sub * 32, 32)])


def _layernorm(h, g, b):
    mu = jnp.mean(h, axis=-1, keepdims=True)
    var = jnp.var(h, axis=-1, keepdims=True)
    return (h - mu) / jnp.sqrt(var + 1e-5) * g + b


def _time_embed(t):
    half = HID // 2
    tf = jnp.clip(t, 0, 999).astype(jnp.float32)
    freqs = jnp.exp(jnp.arange(half, dtype=jnp.float32) * -(math.log(10000.0) / (half - 1)))
    ang = tf[:, None] * freqs[None, :]
    return jnp.concatenate([jnp.sin(ang), jnp.cos(ang)], axis=-1)


def _gatv2(xn, edata, e_tab, loop_e, p):
    xl = xn @ p["Wl"] + p["bl"]
    xr = xn @ p["Wr"] + p["br"]
    att = p["att"].reshape(-1)

    num_q = []
    ex_q = []
    for hh in range(2):
        sl = slice(hh * 128, (hh + 1) * 128)
        out, den = _edge_half(xl[:, sl], xr[:, sl], e_tab[:, sl], att[sl], edata)
        num_q.append((out[0] + out[1])[:N_NODES])
        ex_q.append((den[0] + den[1]).reshape(-1)[:N_NODES * 4].reshape(N_NODES, 4))
    num = jnp.concatenate(num_q, axis=-1).reshape(-1, HEADS, OUTC)
    ex_sum = jnp.concatenate(ex_q, axis=-1)

    xl3 = xl.reshape(-1, HEADS, OUTC)
    xr3 = xr.reshape(-1, HEADS, OUTC)
    m_loop = jax.nn.leaky_relu(xl3 + xr3 + loop_e, 0.2)
    a_loop = jnp.sum(m_loop * p["att"][None], axis=-1)
    ex_loop = jnp.exp(a_loop)
    den = ex_sum + ex_loop
    out = (num + xl3 * ex_loop[..., None]) / den[..., None]
    return out.reshape(-1, HID) + p["bias"]


def kernel(x, edge_index, edge_attr, batch, timesteps, params):
    xi = jnp.clip(x, 0, NODE_DIM - 1)
    ei = jnp.clip(edge_attr, 0, EDGE_DIM - 1).astype(jnp.int32)
    node = params["node_embed"][xi]
    te = _time_embed(timesteps)
    te = jax.nn.gelu(te @ params["tm_W1"] + params["tm_b1"], approximate=False) @ params["tm_W2"] + params["tm_b2"]
    node = node + te[batch]
    src = edge_index[0].astype(jnp.int32)
    dst = edge_index[1].astype(jnp.int32)

    pad = EPAD - N_EDGES
    srcp = jnp.concatenate([src, jnp.zeros((pad,), jnp.int32)])
    dstp = jnp.concatenate([dst, jnp.zeros((pad,), jnp.int32)])
    eip = jnp.concatenate([ei, jnp.zeros((pad,), jnp.int32)])
    edata = jnp.stack([srcp.reshape(-1, CH), dstp.reshape(-1, CH),
                       eip.reshape(-1, CH)], axis=1).reshape(-1)

    mask = (src != dst)
    mf = mask.astype(jnp.float32)
    onehot = jax.nn.one_hot(ei, EDGE_DIM, dtype=jnp.float32) * mf[:, None]
    cnt16 = jax.ops.segment_sum(onehot, dst, num_segments=N_NODES)
    cnt = jnp.sum(cnt16, axis=-1)
    emean = (cnt16 @ params["edge_embed"]) / jnp.maximum(cnt, 1.0)[:, None]

    for lp in params["layers"]:
        e_tab = params["edge_embed"] @ lp["We"]
        loop_e = (emean @ lp["We"]).reshape(-1, HEADS, OUTC)
        attn = _gatv2(node, edata, e_tab, loop_e, lp)
        node = _layernorm(node + attn, lp["n1g"], lp["n1b"])
        h = jax.nn.gelu(node @ lp["fW1"] + lp["fb1"], approximate=False) @ lp["fW2"] + lp["fb2"]
        node = _layernorm(node + h, lp["n2g"], lp["n2b"])

    pred_node = _layernorm(node, params["hn_g"], params["hn_b"]) @ params["hn_W"] + params["hn_bW"]
    ef = (node[src] + node[dst]) * 0.5
    pred_edge = _layernorm(ef, params["he_g"], params["he_b"]) @ params["he_W"] + params["he_bW"]
    return (pred_node, pred_edge)
